# baseline (device time: 180307 ns/iter reference)
import jax
import jax.numpy as jnp
from jax import lax
from jax.experimental import pallas as pl
from jax.experimental.pallas import tpu as pltpu

N_DEV = 8


def kernel(x, Win0, Wout0, Win1, Wout1, Win2, Wout2):
    m_per, d = x.shape
    M = N_DEV * m_per

    def body(x_ref, win0_ref, wout0_ref, win1_ref, wout1_ref, win2_ref,
             wout2_ref, out_ref, xfull_ref, acc_ref, rs_send_ref, rs_recv_ref,
             ag_send_sem, ag_recv_sems, rs_send_sem, rs_recv_sems):
        my = lax.axis_index("i")
        left = lax.rem(my + N_DEV - 1, N_DEV)
        right = lax.rem(my + 1, N_DEV)

        barrier_sem = pltpu.get_barrier_semaphore()
        for nbr in (left, right):
            pl.semaphore_signal(barrier_sem, inc=1, device_id=(nbr,),
                                device_id_type=pl.DeviceIdType.MESH)
        pl.semaphore_wait(barrier_sem, 2)

        def ring_all_gather(parity):
            for h in range(N_DEV - 1):
                c = lax.rem(my + (N_DEV - h), N_DEV) * m_per
                rdma = pltpu.make_async_remote_copy(
                    src_ref=xfull_ref.at[parity, pl.ds(c, m_per), :],
                    dst_ref=xfull_ref.at[parity, pl.ds(c, m_per), :],
                    send_sem=ag_send_sem,
                    recv_sem=ag_recv_sems.at[h],
                    device_id=(right,),
                    device_id_type=pl.DeviceIdType.MESH,
                )
                rdma.start()
                rdma.wait()

        def ring_reduce_scatter():
            for s in range(N_DEV - 1):
                c = lax.rem(my + (N_DEV - 1 - s), N_DEV) * m_per
                val = acc_ref[pl.ds(c, m_per), :]
                if s > 0:
                    val = val + rs_recv_ref[s - 1]
                rs_send_ref[...] = val
                rdma = pltpu.make_async_remote_copy(
                    src_ref=rs_send_ref,
                    dst_ref=rs_recv_ref.at[s],
                    send_sem=rs_send_sem,
                    recv_sem=rs_recv_sems.at[s],
                    device_id=(right,),
                    device_id_type=pl.DeviceIdType.MESH,
                )
                rdma.start()
                rdma.wait()
            return acc_ref[pl.ds(my * m_per, m_per), :] + rs_recv_ref[N_DEV - 2]

        xfull_ref[0, pl.ds(my * m_per, m_per), :] = (
            x_ref[...].astype(jnp.bfloat16))

        layers = ((win0_ref, wout0_ref), (win1_ref, wout1_ref),
                  (win2_ref, wout2_ref))
        for l, (win_ref, wout_ref) in enumerate(layers):
            p = l % 2
            ring_all_gather(p)
            xf = xfull_ref[p]
            h = jnp.maximum(
                jnp.dot(xf, win_ref[...].astype(jnp.bfloat16),
                        preferred_element_type=jnp.float32), 0.0,
            ).astype(jnp.bfloat16)
            acc_ref[...] = jnp.dot(h, wout_ref[...].astype(jnp.bfloat16),
                                   preferred_element_type=jnp.float32)
            x_next = ring_reduce_scatter()
            if l < len(layers) - 1:
                xfull_ref[1 - p, pl.ds(my * m_per, m_per), :] = (
                    x_next.astype(jnp.bfloat16))
            else:
                out_ref[...] = x_next

    return pl.pallas_call(
        body,
        out_shape=jax.ShapeDtypeStruct((m_per, d), jnp.float32),
        in_specs=[pl.BlockSpec(memory_space=pltpu.VMEM)] * 7,
        out_specs=pl.BlockSpec(memory_space=pltpu.VMEM),
        scratch_shapes=[
            pltpu.VMEM((2, M, d), jnp.bfloat16),
            pltpu.VMEM((M, d), jnp.float32),
            pltpu.VMEM((m_per, d), jnp.float32),
            pltpu.VMEM((N_DEV - 1, m_per, d), jnp.float32),
            pltpu.SemaphoreType.DMA,
            pltpu.SemaphoreType.DMA((N_DEV - 1,)),
            pltpu.SemaphoreType.DMA,
            pltpu.SemaphoreType.DMA((N_DEV - 1,)),
        ],
        compiler_params=pltpu.CompilerParams(collective_id=0),
    )(x, Win0, Wout0, Win1, Wout1, Win2, Wout2)


# device time: 109279 ns/iter; 1.6500x vs baseline; 1.6500x over previous
import jax
import jax.numpy as jnp
from jax import lax
from jax.experimental import pallas as pl
from jax.experimental.pallas import tpu as pltpu

N_DEV = 8


def kernel(x, Win0, Wout0, Win1, Wout1, Win2, Wout2):
    m_per, d = x.shape
    M = N_DEV * m_per

    def body(x_ref, win0_ref, wout0_ref, win1_ref, wout1_ref, win2_ref,
             wout2_ref, out_ref, xfull_ref, acc_ref, rs_send_ref, rs_recv_ref,
             ag_send_sems, ag_recv_sems, rs_send_sems, rs_recv_sems):
        my = lax.axis_index("i")

        barrier_sem = pltpu.get_barrier_semaphore()
        for bit in (1, 2, 4):
            pl.semaphore_signal(
                barrier_sem, inc=1,
                device_id=(lax.bitwise_xor(my, bit),),
                device_id_type=pl.DeviceIdType.MESH)
        pl.semaphore_wait(barrier_sem, 3)

        xfull_ref[pl.ds(my * m_per, m_per), :] = (
            x_ref[...].astype(jnp.bfloat16))

        def all_gather():
            for s in range(3):
                bit = 1 << s
                sz = bit * m_per
                partner = lax.bitwise_xor(my, bit)
                ac = lax.bitwise_and(my, (~(bit - 1)) & 7) * m_per
                rdma = pltpu.make_async_remote_copy(
                    src_ref=xfull_ref.at[pl.ds(ac, sz), :],
                    dst_ref=xfull_ref.at[pl.ds(ac, sz), :],
                    send_sem=ag_send_sems.at[s],
                    recv_sem=ag_recv_sems.at[s],
                    device_id=(partner,),
                    device_id_type=pl.DeviceIdType.MESH,
                )
                rdma.start()
                rdma.wait()

        def reduce_scatter():
            off = 0
            for t, bit in enumerate((4, 2, 1)):
                sz = bit * m_per
                partner = lax.bitwise_xor(my, bit)
                kc = lax.bitwise_and(my, (~(bit - 1)) & 7)
                sc_row = lax.bitwise_xor(kc, bit) * m_per
                kc_row = kc * m_per
                rs_send_ref[pl.ds(0, sz), :] = (
                    acc_ref[pl.ds(sc_row, sz), :].astype(jnp.bfloat16))
                rdma = pltpu.make_async_remote_copy(
                    src_ref=rs_send_ref.at[pl.ds(0, sz), :],
                    dst_ref=rs_recv_ref.at[pl.ds(off, sz), :],
                    send_sem=rs_send_sems.at[t],
                    recv_sem=rs_recv_sems.at[t],
                    device_id=(partner,),
                    device_id_type=pl.DeviceIdType.MESH,
                )
                rdma.start()
                rdma.wait()
                summed = (acc_ref[pl.ds(kc_row, sz), :]
                          + rs_recv_ref[pl.ds(off, sz), :].astype(jnp.float32))
                if t < 2:
                    acc_ref[pl.ds(kc_row, sz), :] = summed
                else:
                    out_ref[...] = summed
                off += sz

        layers = ((win0_ref, wout0_ref), (win1_ref, wout1_ref),
                  (win2_ref, wout2_ref))
        for l, (win_ref, wout_ref) in enumerate(layers):
            all_gather()
            xf = xfull_ref[...]
            h = jnp.maximum(
                jnp.dot(xf, win_ref[...].astype(jnp.bfloat16),
                        preferred_element_type=jnp.float32), 0.0,
            ).astype(jnp.bfloat16)
            acc_ref[...] = jnp.dot(h, wout_ref[...].astype(jnp.bfloat16),
                                   preferred_element_type=jnp.float32)
            reduce_scatter()
            if l < len(layers) - 1:
                xfull_ref[pl.ds(my * m_per, m_per), :] = (
                    out_ref[...].astype(jnp.bfloat16))

    return pl.pallas_call(
        body,
        out_shape=jax.ShapeDtypeStruct((m_per, d), jnp.float32),
        in_specs=[pl.BlockSpec(memory_space=pltpu.VMEM)] * 7,
        out_specs=pl.BlockSpec(memory_space=pltpu.VMEM),
        scratch_shapes=[
            pltpu.VMEM((M, d), jnp.bfloat16),
            pltpu.VMEM((M, d), jnp.float32),
            pltpu.VMEM((4 * m_per, d), jnp.bfloat16),
            pltpu.VMEM((7 * m_per, d), jnp.bfloat16),
            pltpu.SemaphoreType.DMA((3,)),
            pltpu.SemaphoreType.DMA((3,)),
            pltpu.SemaphoreType.DMA((3,)),
            pltpu.SemaphoreType.DMA((3,)),
        ],
        compiler_params=pltpu.CompilerParams(collective_id=0),
    )(x, Win0, Wout0, Win1, Wout1, Win2, Wout2)


# device time: 86972 ns/iter; 2.0732x vs baseline; 1.2565x over previous
import jax
import jax.numpy as jnp
from jax import lax
from jax.experimental import pallas as pl
from jax.experimental.pallas import tpu as pltpu

N_DEV = 8

AG_BITS_1 = (1, 2, 4)
AG_BITS_2 = (2, 4, 1)
RS_BITS_1 = (4, 2, 1)
RS_BITS_2 = (1, 4, 2)


def kernel(x, Win0, Wout0, Win1, Wout1, Win2, Wout2):
    m_per, d = x.shape
    H = m_per // 2
    MH = N_DEV * H

    def body(x_ref, win0_ref, wout0_ref, win1_ref, wout1_ref, win2_ref,
             wout2_ref, out_ref, xf1_ref, xf2_ref, acc1_ref, acc2_ref,
             snd1_ref, snd2_ref, rcv1_ref, rcv2_ref,
             ag1_s, ag1_r, ag2_s, ag2_r, rs1_s, rs1_r, rs2_s, rs2_r):
        my = lax.axis_index("i")
        pmy = lax.bitwise_or(
            lax.shift_right_logical(my, 1),
            lax.shift_left(lax.bitwise_and(my, 1), 2))

        barrier_sem = pltpu.get_barrier_semaphore()
        for bit in (1, 2, 4):
            pl.semaphore_signal(
                barrier_sem, inc=1,
                device_id=(lax.bitwise_xor(my, bit),),
                device_id_type=pl.DeviceIdType.MESH)
        pl.semaphore_wait(barrier_sem, 3)

        def row(i):
            return pl.multiple_of(i, H)

        def exchange(xf_ref, idx, cbit, r, sz, s_sems, r_sems):
            a = row(lax.bitwise_and(idx, (~((1 << r) - 1)) & 7) * H)
            return pltpu.make_async_remote_copy(
                src_ref=xf_ref.at[pl.ds(a, sz), :],
                dst_ref=xf_ref.at[pl.ds(a, sz), :],
                send_sem=s_sems.at[r],
                recv_sem=r_sems.at[r],
                device_id=(lax.bitwise_xor(my, cbit),),
                device_id_type=pl.DeviceIdType.MESH,
            )

        def compute_rows(win_ref, wout_ref, xf_ref, acc_ref, start, nrows):
            start = row(start)
            xh = xf_ref[pl.ds(start, nrows), :]
            h = jnp.maximum(
                jnp.dot(xh, win_ref[...].astype(jnp.bfloat16),
                        preferred_element_type=jnp.float32), 0.0,
            ).astype(jnp.bfloat16)
            acc_ref[pl.ds(start, nrows), :] = jnp.dot(
                h, wout_ref[...].astype(jnp.bfloat16),
                preferred_element_type=jnp.float32)

        def rs_step(t, idx, cbit, acc_ref, snd_ref, rcv_ref, s_sems, r_sems,
                    off):
            bit = (4, 2, 1)[t]
            sz = bit * H
            kc = lax.bitwise_and(idx, (~(bit - 1)) & 7)
            sc = row(lax.bitwise_xor(kc, bit) * H)
            snd_ref[pl.ds(0, sz), :] = (
                acc_ref[pl.ds(sc, sz), :].astype(jnp.bfloat16))
            rdma = pltpu.make_async_remote_copy(
                src_ref=snd_ref.at[pl.ds(0, sz), :],
                dst_ref=rcv_ref.at[pl.ds(off, sz), :],
                send_sem=s_sems.at[t],
                recv_sem=r_sems.at[t],
                device_id=(lax.bitwise_xor(my, cbit),),
                device_id_type=pl.DeviceIdType.MESH,
            )
            rdma.start()
            return rdma, row(kc * H), sz

        xf1_ref[pl.ds(row(my * H), H), :] = x_ref[0:H, :].astype(jnp.bfloat16)
        xf2_ref[pl.ds(row(pmy * H), H), :] = x_ref[H:m_per, :].astype(jnp.bfloat16)

        layers = ((win0_ref, wout0_ref), (win1_ref, wout1_ref),
                  (win2_ref, wout2_ref))
        for l, (win_ref, wout_ref) in enumerate(layers):
            for r in range(3):
                sz = (1 << r) * H
                r1 = exchange(xf1_ref, my, AG_BITS_1[r], r, sz, ag1_s, ag1_r)
                r2 = exchange(xf2_ref, pmy, AG_BITS_2[r], r, sz, ag2_s, ag2_r)
                r1.start()
                r2.start()
                r1.wait()
                r2.wait()
            a1 = row(lax.bitwise_and(my, 4) * H)
            a2 = row(lax.bitwise_and(pmy, 4) * H)
            compute_rows(win_ref, wout_ref, xf1_ref, acc1_ref, a1, 4 * H)
            compute_rows(win_ref, wout_ref, xf2_ref, acc2_ref, a2, 4 * H)
            o1 = row(lax.bitwise_xor(a1, 4 * H))
            o2 = row(lax.bitwise_xor(a2, 4 * H))
            compute_rows(win_ref, wout_ref, xf1_ref, acc1_ref, o1, 4 * H)
            compute_rows(win_ref, wout_ref, xf2_ref, acc2_ref, o2, 4 * H)

            off = 0
            for t in range(3):
                d1, k1, sz = rs_step(t, my, RS_BITS_1[t], acc1_ref, snd1_ref,
                                     rcv1_ref, rs1_s, rs1_r, off)
                d2, k2, _ = rs_step(t, pmy, RS_BITS_2[t], acc2_ref, snd2_ref,
                                    rcv2_ref, rs2_s, rs2_r, off)
                d1.wait()
                d2.wait()
                s1 = (acc1_ref[pl.ds(k1, sz), :]
                      + rcv1_ref[pl.ds(off, sz), :].astype(jnp.float32))
                s2 = (acc2_ref[pl.ds(k2, sz), :]
                      + rcv2_ref[pl.ds(off, sz), :].astype(jnp.float32))
                if t < 2:
                    acc1_ref[pl.ds(k1, sz), :] = s1
                    acc2_ref[pl.ds(k2, sz), :] = s2
                else:
                    out_ref[0:H, :] = s1
                    out_ref[H:m_per, :] = s2
                    if l < len(layers) - 1:
                        xf1_ref[pl.ds(row(my * H), H), :] = s1.astype(jnp.bfloat16)
                        xf2_ref[pl.ds(row(pmy * H), H), :] = s2.astype(jnp.bfloat16)
                off += sz

    return pl.pallas_call(
        body,
        out_shape=jax.ShapeDtypeStruct((m_per, d), jnp.float32),
        in_specs=[pl.BlockSpec(memory_space=pltpu.VMEM)] * 7,
        out_specs=pl.BlockSpec(memory_space=pltpu.VMEM),
        scratch_shapes=[
            pltpu.VMEM((MH, d), jnp.bfloat16),
            pltpu.VMEM((MH, d), jnp.bfloat16),
            pltpu.VMEM((MH, d), jnp.float32),
            pltpu.VMEM((MH, d), jnp.float32),
            pltpu.VMEM((4 * H, d), jnp.bfloat16),
            pltpu.VMEM((4 * H, d), jnp.bfloat16),
            pltpu.VMEM((7 * H, d), jnp.bfloat16),
            pltpu.VMEM((7 * H, d), jnp.bfloat16),
            pltpu.SemaphoreType.DMA((3,)),
            pltpu.SemaphoreType.DMA((3,)),
            pltpu.SemaphoreType.DMA((3,)),
            pltpu.SemaphoreType.DMA((3,)),
            pltpu.SemaphoreType.DMA((3,)),
            pltpu.SemaphoreType.DMA((3,)),
            pltpu.SemaphoreType.DMA((3,)),
            pltpu.SemaphoreType.DMA((3,)),
        ],
        compiler_params=pltpu.CompilerParams(collective_id=0),
    )(x, Win0, Wout0, Win1, Wout1, Win2, Wout2)


# device time: 85241 ns/iter; 2.1153x vs baseline; 1.0203x over previous
import jax
import jax.numpy as jnp
from jax import lax
from jax.experimental import pallas as pl
from jax.experimental.pallas import tpu as pltpu

N_DEV = 8

AG_BITS_1 = (1, 2, 4)
AG_BITS_2 = (2, 4, 1)
RS_BITS_1 = (4, 2, 1)
RS_BITS_2 = (1, 4, 2)


def kernel(x, Win0, Wout0, Win1, Wout1, Win2, Wout2):
    m_per, d = x.shape
    H = m_per // 2
    MH = N_DEV * H

    def body(x_ref, win0_ref, wout0_ref, win1_ref, wout1_ref, win2_ref,
             wout2_ref, out_ref, xf1_ref, xf2_ref, acc1_ref, acc2_ref,
             snd1_ref, snd2_ref, rcv1_ref, rcv2_ref,
             ag1_s, ag1_r, ag2_s, ag2_r, rs1_s, rs1_r, rs2_s, rs2_r):
        my = lax.axis_index("i")
        pmy = lax.bitwise_or(
            lax.shift_right_logical(my, 1),
            lax.shift_left(lax.bitwise_and(my, 1), 2))

        barrier_sem = pltpu.get_barrier_semaphore()
        for bit in (1, 2, 4):
            pl.semaphore_signal(
                barrier_sem, inc=1,
                device_id=(lax.bitwise_xor(my, bit),),
                device_id_type=pl.DeviceIdType.MESH)
        pl.semaphore_wait(barrier_sem, 3)

        def row(i):
            return pl.multiple_of(i, H)

        def exchange(xf_ref, idx, cbit, r, sz, s_sems, r_sems):
            a = row(lax.bitwise_and(idx, (~((1 << r) - 1)) & 7) * H)
            return pltpu.make_async_remote_copy(
                src_ref=xf_ref.at[pl.ds(a, sz), :],
                dst_ref=xf_ref.at[pl.ds(a, sz), :],
                send_sem=s_sems.at[r],
                recv_sem=r_sems.at[r],
                device_id=(lax.bitwise_xor(my, cbit),),
                device_id_type=pl.DeviceIdType.MESH,
            )

        def compute_rows(win_ref, wout_ref, xf_ref, acc_ref, start, nrows):
            start = row(start)
            xh = xf_ref[pl.ds(start, nrows), :]
            h = jnp.maximum(
                jnp.dot(xh, win_ref[...].astype(jnp.bfloat16),
                        preferred_element_type=jnp.float32), 0.0,
            ).astype(jnp.bfloat16)
            acc_ref[pl.ds(start, nrows), :] = jnp.dot(
                h, wout_ref[...].astype(jnp.bfloat16),
                preferred_element_type=jnp.float32)

        def rs_step(t, idx, cbit, acc_ref, snd_ref, rcv_ref, s_sems, r_sems,
                    off):
            bit = (4, 2, 1)[t]
            sz = bit * H
            kc = lax.bitwise_and(idx, (~(bit - 1)) & 7)
            sc = row(lax.bitwise_xor(kc, bit) * H)
            snd_ref[pl.ds(0, sz), :] = (
                acc_ref[pl.ds(sc, sz), :].astype(jnp.bfloat16))
            rdma = pltpu.make_async_remote_copy(
                src_ref=snd_ref.at[pl.ds(0, sz), :],
                dst_ref=rcv_ref.at[pl.ds(off, sz), :],
                send_sem=s_sems.at[t],
                recv_sem=r_sems.at[t],
                device_id=(lax.bitwise_xor(my, cbit),),
                device_id_type=pl.DeviceIdType.MESH,
            )
            rdma.start()
            return rdma, row(kc * H), sz

        xf1_ref[pl.ds(row(my * H), H), :] = x_ref[0:H, :].astype(jnp.bfloat16)
        xf2_ref[pl.ds(row(pmy * H), H), :] = x_ref[H:m_per, :].astype(jnp.bfloat16)

        layers = ((win0_ref, wout0_ref), (win1_ref, wout1_ref),
                  (win2_ref, wout2_ref))
        for l, (win_ref, wout_ref) in enumerate(layers):
            for r in range(3):
                sz = (1 << r) * H
                r1 = exchange(xf1_ref, my, AG_BITS_1[r], r, sz, ag1_s, ag1_r)
                r2 = exchange(xf2_ref, pmy, AG_BITS_2[r], r, sz, ag2_s, ag2_r)
                r1.start()
                r2.start()
                if r < 2:
                    r1.wait()
                    r2.wait()
            a1 = row(lax.bitwise_and(my, 4) * H)
            a2 = row(lax.bitwise_and(pmy, 4) * H)
            compute_rows(win_ref, wout_ref, xf1_ref, acc1_ref, a1, 4 * H)
            compute_rows(win_ref, wout_ref, xf2_ref, acc2_ref, a2, 4 * H)
            r1.wait()
            r2.wait()
            o1 = row(lax.bitwise_xor(a1, 4 * H))
            o2 = row(lax.bitwise_xor(a2, 4 * H))
            compute_rows(win_ref, wout_ref, xf1_ref, acc1_ref, o1, 4 * H)
            compute_rows(win_ref, wout_ref, xf2_ref, acc2_ref, o2, 4 * H)

            off = 0
            for t in range(3):
                d1, k1, sz = rs_step(t, my, RS_BITS_1[t], acc1_ref, snd1_ref,
                                     rcv1_ref, rs1_s, rs1_r, off)
                d2, k2, _ = rs_step(t, pmy, RS_BITS_2[t], acc2_ref, snd2_ref,
                                    rcv2_ref, rs2_s, rs2_r, off)
                d1.wait()
                d2.wait()
                s1 = (acc1_ref[pl.ds(k1, sz), :]
                      + rcv1_ref[pl.ds(off, sz), :].astype(jnp.float32))
                s2 = (acc2_ref[pl.ds(k2, sz), :]
                      + rcv2_ref[pl.ds(off, sz), :].astype(jnp.float32))
                if t < 2:
                    acc1_ref[pl.ds(k1, sz), :] = s1
                    acc2_ref[pl.ds(k2, sz), :] = s2
                else:
                    out_ref[0:H, :] = s1
                    out_ref[H:m_per, :] = s2
                    if l < len(layers) - 1:
                        xf1_ref[pl.ds(row(my * H), H), :] = s1.astype(jnp.bfloat16)
                        xf2_ref[pl.ds(row(pmy * H), H), :] = s2.astype(jnp.bfloat16)
                off += sz

    return pl.pallas_call(
        body,
        out_shape=jax.ShapeDtypeStruct((m_per, d), jnp.float32),
        in_specs=[pl.BlockSpec(memory_space=pltpu.VMEM)] * 7,
        out_specs=pl.BlockSpec(memory_space=pltpu.VMEM),
        scratch_shapes=[
            pltpu.VMEM((MH, d), jnp.bfloat16),
            pltpu.VMEM((MH, d), jnp.bfloat16),
            pltpu.VMEM((MH, d), jnp.float32),
            pltpu.VMEM((MH, d), jnp.float32),
            pltpu.VMEM((4 * H, d), jnp.bfloat16),
            pltpu.VMEM((4 * H, d), jnp.bfloat16),
            pltpu.VMEM((7 * H, d), jnp.bfloat16),
            pltpu.VMEM((7 * H, d), jnp.bfloat16),
            pltpu.SemaphoreType.DMA((3,)),
            pltpu.SemaphoreType.DMA((3,)),
            pltpu.SemaphoreType.DMA((3,)),
            pltpu.SemaphoreType.DMA((3,)),
            pltpu.SemaphoreType.DMA((3,)),
            pltpu.SemaphoreType.DMA((3,)),
            pltpu.SemaphoreType.DMA((3,)),
            pltpu.SemaphoreType.DMA((3,)),
        ],
        compiler_params=pltpu.CompilerParams(collective_id=0),
    )(x, Win0, Wout0, Win1, Wout1, Win2, Wout2)


# device time: 80072 ns/iter; 2.2518x vs baseline; 1.0646x over previous
import jax
import jax.numpy as jnp
from jax import lax
from jax.experimental import pallas as pl
from jax.experimental.pallas import tpu as pltpu

N_DEV = 8

AG_BITS_1 = (1, 2, 4)
AG_BITS_2 = (2, 4, 1)
RS_BITS_1 = (4, 2, 1)
RS_BITS_2 = (1, 4, 2)


def kernel(x, Win0, Wout0, Win1, Wout1, Win2, Wout2):
    m_per, d = x.shape
    hid = Win0.shape[1]
    H = m_per // 2
    MH = N_DEV * H

    def body(x_ref, win0_ref, wout0_ref, win1_ref, wout1_ref, win2_ref,
             wout2_ref, out_ref, xf1_ref, xf2_ref, acc1_ref, acc2_ref,
             snd1_ref, snd2_ref, rcv1_ref, rcv2_ref, wbin_ref, wbout_ref,
             ag1_s, ag1_r, ag2_s, ag2_r, rs1_s, rs1_r, rs2_s, rs2_r):
        my = lax.axis_index("i")
        pmy = lax.bitwise_or(
            lax.shift_right_logical(my, 1),
            lax.shift_left(lax.bitwise_and(my, 1), 2))

        def row(i):
            return pl.multiple_of(i, H)

        xf1_ref[pl.ds(row(my * H), H), :] = x_ref[0:H, :].astype(jnp.bfloat16)
        xf2_ref[pl.ds(row(pmy * H), H), :] = (
            x_ref[H:m_per, :].astype(jnp.bfloat16))

        barrier_sem = pltpu.get_barrier_semaphore()
        for bit in (1, 2, 4):
            pl.semaphore_signal(
                barrier_sem, inc=1,
                device_id=(lax.bitwise_xor(my, bit),),
                device_id_type=pl.DeviceIdType.MESH)
        pl.semaphore_wait(barrier_sem, 3)

        def exchange(xf_ref, idx, cbit, r, sz, s_sems, r_sems):
            a = row(lax.bitwise_and(idx, (~((1 << r) - 1)) & 7) * H)
            rdma = pltpu.make_async_remote_copy(
                src_ref=xf_ref.at[pl.ds(a, sz), :],
                dst_ref=xf_ref.at[pl.ds(a, sz), :],
                send_sem=s_sems.at[r],
                recv_sem=r_sems.at[r],
                device_id=(lax.bitwise_xor(my, cbit),),
                device_id_type=pl.DeviceIdType.MESH,
            )
            rdma.start()
            return rdma

        def cast_weights(win_ref, wout_ref):
            wbin_ref[...] = win_ref[...].astype(jnp.bfloat16)
            wbout_ref[...] = wout_ref[...].astype(jnp.bfloat16)

        def compute_rows(xf_ref, acc_ref, start, nrows):
            xh = xf_ref[pl.ds(row(start), nrows), :]
            h = jnp.maximum(
                jnp.dot(xh, wbin_ref[...],
                        preferred_element_type=jnp.float32), 0.0,
            ).astype(jnp.bfloat16)
            acc_ref[pl.ds(row(start), nrows), :] = jnp.dot(
                h, wbout_ref[...], preferred_element_type=jnp.float32)

        def rs_start(t, idx, cbit, acc_ref, snd_ref, rcv_ref, s_sems, r_sems,
                     off, sz):
            bit = sz // H
            kc = lax.bitwise_and(idx, (~(bit - 1)) & 7)
            sc = row(lax.bitwise_xor(kc, bit) * H)
            snd_ref[pl.ds(0, sz), :] = (
                acc_ref[pl.ds(sc, sz), :].astype(jnp.bfloat16))
            rdma = pltpu.make_async_remote_copy(
                src_ref=snd_ref.at[pl.ds(0, sz), :],
                dst_ref=rcv_ref.at[pl.ds(off, sz), :],
                send_sem=s_sems.at[t],
                recv_sem=r_sems.at[t],
                device_id=(lax.bitwise_xor(my, cbit),),
                device_id_type=pl.DeviceIdType.MESH,
            )
            rdma.start()
            return rdma, row(kc * H)

        def bdy_start(idx, cbit, acc_ref, snd_ref, rcv_ref, s_sems, r_sems,
                      si, off):
            kb = row(lax.bitwise_and(idx, 6) * H)
            snd_ref[pl.ds(0, 2 * H), :] = (
                acc_ref[pl.ds(kb, 2 * H), :].astype(jnp.bfloat16))
            rdma = pltpu.make_async_remote_copy(
                src_ref=snd_ref.at[pl.ds(0, 2 * H), :],
                dst_ref=rcv_ref.at[pl.ds(off, 2 * H), :],
                send_sem=s_sems.at[si],
                recv_sem=r_sems.at[si],
                device_id=(lax.bitwise_xor(my, cbit),),
                device_id_type=pl.DeviceIdType.MESH,
            )
            rdma.start()
            return rdma, kb

        layers = ((win0_ref, wout0_ref), (win1_ref, wout1_ref),
                  (win2_ref, wout2_ref))
        last = len(layers) - 1
        for l, (win_ref, wout_ref) in enumerate(layers):
            if l == 0:
                r1 = exchange(xf1_ref, my, AG_BITS_1[0], 0, H, ag1_s, ag1_r)
                r2 = exchange(xf2_ref, pmy, AG_BITS_2[0], 0, H, ag2_s, ag2_r)
                cast_weights(win_ref, wout_ref)
                r1.wait()
                r2.wait()
            r1 = exchange(xf1_ref, my, AG_BITS_1[1], 1, 2 * H, ag1_s, ag1_r)
            r2 = exchange(xf2_ref, pmy, AG_BITS_2[1], 1, 2 * H, ag2_s, ag2_r)
            c1 = row(lax.bitwise_and(my, 6) * H)
            c2 = row(lax.bitwise_and(pmy, 6) * H)
            compute_rows(xf1_ref, acc1_ref, c1, 2 * H)
            compute_rows(xf2_ref, acc2_ref, c2, 2 * H)
            r1.wait()
            r2.wait()
            r1 = exchange(xf1_ref, my, AG_BITS_1[2], 2, 4 * H, ag1_s, ag1_r)
            r2 = exchange(xf2_ref, pmy, AG_BITS_2[2], 2, 4 * H, ag2_s, ag2_r)
            compute_rows(xf1_ref, acc1_ref, lax.bitwise_xor(c1, 2 * H), 2 * H)
            compute_rows(xf2_ref, acc2_ref, lax.bitwise_xor(c2, 2 * H), 2 * H)
            r1.wait()
            r2.wait()
            a1 = row(lax.bitwise_and(my, 4) * H)
            a2 = row(lax.bitwise_and(pmy, 4) * H)
            compute_rows(xf1_ref, acc1_ref, lax.bitwise_xor(a1, 4 * H), 4 * H)
            compute_rows(xf2_ref, acc2_ref, lax.bitwise_xor(a2, 4 * H), 4 * H)

            d1, k1 = rs_start(0, my, RS_BITS_1[0], acc1_ref, snd1_ref,
                              rcv1_ref, rs1_s, rs1_r, 0, 4 * H)
            d2, k2 = rs_start(0, pmy, RS_BITS_2[0], acc2_ref, snd2_ref,
                              rcv2_ref, rs2_s, rs2_r, 0, 4 * H)
            if l < last:
                cast_weights(*layers[l + 1])
            d1.wait()
            d2.wait()
            acc1_ref[pl.ds(k1, 4 * H), :] = (
                acc1_ref[pl.ds(k1, 4 * H), :]
                + rcv1_ref[pl.ds(0, 4 * H), :].astype(jnp.float32))
            acc2_ref[pl.ds(k2, 4 * H), :] = (
                acc2_ref[pl.ds(k2, 4 * H), :]
                + rcv2_ref[pl.ds(0, 4 * H), :].astype(jnp.float32))
            d1, k1 = rs_start(1, my, RS_BITS_1[1], acc1_ref, snd1_ref,
                              rcv1_ref, rs1_s, rs1_r, 4 * H, 2 * H)
            d2, k2 = rs_start(1, pmy, RS_BITS_2[1], acc2_ref, snd2_ref,
                              rcv2_ref, rs2_s, rs2_r, 4 * H, 2 * H)
            d1.wait()
            d2.wait()
            acc1_ref[pl.ds(k1, 2 * H), :] = (
                acc1_ref[pl.ds(k1, 2 * H), :]
                + rcv1_ref[pl.ds(4 * H, 2 * H), :].astype(jnp.float32))
            acc2_ref[pl.ds(k2, 2 * H), :] = (
                acc2_ref[pl.ds(k2, 2 * H), :]
                + rcv2_ref[pl.ds(4 * H, 2 * H), :].astype(jnp.float32))

            if l < last:
                si = 2 + (l % 2)
                boff = (6 + 2 * (l % 2)) * H
                d1, k1 = bdy_start(my, RS_BITS_1[2], acc1_ref, snd1_ref,
                                   rcv1_ref, rs1_s, rs1_r, si, boff)
                d2, k2 = bdy_start(pmy, RS_BITS_2[2], acc2_ref, snd2_ref,
                                   rcv2_ref, rs2_s, rs2_r, si, boff)
                d1.wait()
                d2.wait()
                xf1_ref[pl.ds(k1, 2 * H), :] = (
                    acc1_ref[pl.ds(k1, 2 * H), :]
                    + rcv1_ref[pl.ds(boff, 2 * H), :].astype(jnp.float32)
                ).astype(jnp.bfloat16)
                xf2_ref[pl.ds(k2, 2 * H), :] = (
                    acc2_ref[pl.ds(k2, 2 * H), :]
                    + rcv2_ref[pl.ds(boff, 2 * H), :].astype(jnp.float32)
                ).astype(jnp.bfloat16)
            else:
                d1, k1 = rs_start(2, my, RS_BITS_1[2], acc1_ref, snd1_ref,
                                  rcv1_ref, rs1_s, rs1_r, 6 * H, H)
                d2, k2 = rs_start(2, pmy, RS_BITS_2[2], acc2_ref, snd2_ref,
                                  rcv2_ref, rs2_s, rs2_r, 6 * H, H)
                d1.wait()
                d2.wait()
                out_ref[0:H, :] = (
                    acc1_ref[pl.ds(k1, H), :]
                    + rcv1_ref[pl.ds(6 * H, H), :].astype(jnp.float32))
                out_ref[H:m_per, :] = (
                    acc2_ref[pl.ds(k2, H), :]
                    + rcv2_ref[pl.ds(6 * H, H), :].astype(jnp.float32))

    return pl.pallas_call(
        body,
        out_shape=jax.ShapeDtypeStruct((m_per, d), jnp.float32),
        in_specs=[pl.BlockSpec(memory_space=pltpu.VMEM)] * 7,
        out_specs=pl.BlockSpec(memory_space=pltpu.VMEM),
        scratch_shapes=[
            pltpu.VMEM((MH, d), jnp.bfloat16),
            pltpu.VMEM((MH, d), jnp.bfloat16),
            pltpu.VMEM((MH, d), jnp.float32),
            pltpu.VMEM((MH, d), jnp.float32),
            pltpu.VMEM((4 * H, d), jnp.bfloat16),
            pltpu.VMEM((4 * H, d), jnp.bfloat16),
            pltpu.VMEM((10 * H, d), jnp.bfloat16),
            pltpu.VMEM((10 * H, d), jnp.bfloat16),
            pltpu.VMEM((d, hid), jnp.bfloat16),
            pltpu.VMEM((hid, d), jnp.bfloat16),
            pltpu.SemaphoreType.DMA((3,)),
            pltpu.SemaphoreType.DMA((3,)),
            pltpu.SemaphoreType.DMA((3,)),
            pltpu.SemaphoreType.DMA((3,)),
            pltpu.SemaphoreType.DMA((4,)),
            pltpu.SemaphoreType.DMA((4,)),
            pltpu.SemaphoreType.DMA((4,)),
            pltpu.SemaphoreType.DMA((4,)),
        ],
        compiler_params=pltpu.CompilerParams(collective_id=0),
    )(x, Win0, Wout0, Win1, Wout1, Win2, Wout2)


# device time: 71267 ns/iter; 2.5300x vs baseline; 1.1235x over previous
import jax
import jax.numpy as jnp
from jax import lax
from jax.experimental import pallas as pl
from jax.experimental.pallas import tpu as pltpu

N_DEV = 8


def kernel(x, Win0, Wout0, Win1, Wout1, Win2, Wout2):
    m_per, d = x.shape
    hid = Win0.shape[1]
    H = m_per // 2
    MH = N_DEV * H

    def body(x_ref, win0_ref, wout0_ref, win1_ref, wout1_ref, win2_ref,
             wout2_ref, out_ref, xf1_ref, xf2_ref, acc1_ref, acc2_ref,
             snd1_ref, snd2_ref, rcv1_ref, rcv2_ref, wbin_ref, wbout_ref,
             ag1_s, ag1_r, ag2_s, ag2_r, rs1_s, rs1_r, rs2_s, rs2_r):
        my = lax.axis_index("i")

        def p_x(m):
            return lax.bitwise_xor(m, 1)

        def p_y(m):
            return lax.add(lax.bitwise_and(m, 4),
                           lax.sub(3, lax.bitwise_and(m, 3)))

        def p_z(m):
            return lax.bitwise_xor(m, 4)

        ybit = lax.bitwise_and(lax.shift_right_logical(my, 1), 1)
        qmy = lax.bitwise_or(
            lax.bitwise_or(ybit, lax.shift_right_logical(
                lax.bitwise_and(my, 4), 1)),
            lax.shift_left(lax.bitwise_xor(lax.bitwise_and(my, 1), ybit), 2))

        AG_P_1 = (p_x, p_y, p_z)
        AG_P_2 = (p_y, p_z, p_x)
        RS_P_1 = (p_z, p_y, p_x)
        RS_P_2 = (p_x, p_z, p_y)

        def row(i):
            return pl.multiple_of(i, H)

        xf1_ref[pl.ds(row(my * H), H), :] = x_ref[0:H, :].astype(jnp.bfloat16)
        xf2_ref[pl.ds(row(qmy * H), H), :] = (
            x_ref[H:m_per, :].astype(jnp.bfloat16))

        barrier_sem = pltpu.get_barrier_semaphore()
        for pf in (p_x, p_y, p_z):
            pl.semaphore_signal(
                barrier_sem, inc=1, device_id=(pf(my),),
                device_id_type=pl.DeviceIdType.MESH)
        pl.semaphore_wait(barrier_sem, 3)

        def exchange(xf_ref, idx, partner, r, sz, s_sems, r_sems):
            a = row(lax.bitwise_and(idx, (~((1 << r) - 1)) & 7) * H)
            rdma = pltpu.make_async_remote_copy(
                src_ref=xf_ref.at[pl.ds(a, sz), :],
                dst_ref=xf_ref.at[pl.ds(a, sz), :],
                send_sem=s_sems.at[r],
                recv_sem=r_sems.at[r],
                device_id=(partner,),
                device_id_type=pl.DeviceIdType.MESH,
            )
            rdma.start()
            return rdma

        def cast_weights(win_ref, wout_ref):
            wbin_ref[...] = win_ref[...].astype(jnp.bfloat16)
            wbout_ref[...] = wout_ref[...].astype(jnp.bfloat16)

        def compute_rows(xf_ref, acc_ref, start, nrows):
            xh = xf_ref[pl.ds(row(start), nrows), :]
            h = jnp.maximum(
                jnp.dot(xh, wbin_ref[...],
                        preferred_element_type=jnp.float32), 0.0,
            ).astype(jnp.bfloat16)
            acc_ref[pl.ds(row(start), nrows), :] = jnp.dot(
                h, wbout_ref[...], preferred_element_type=jnp.float32)

        def send_block(src_row, sz, partner, acc_ref, snd_ref, rcv_ref,
                       s_sems, r_sems, si, soff, doff):
            snd_ref[pl.ds(soff, sz), :] = (
                acc_ref[pl.ds(row(src_row), sz), :].astype(jnp.bfloat16))
            rdma = pltpu.make_async_remote_copy(
                src_ref=snd_ref.at[pl.ds(soff, sz), :],
                dst_ref=rcv_ref.at[pl.ds(doff, sz), :],
                send_sem=s_sems.at[si],
                recv_sem=r_sems.at[si],
                device_id=(partner,),
                device_id_type=pl.DeviceIdType.MESH,
            )
            rdma.start()
            return rdma

        def add_in(acc_ref, rcv_ref, dst_row, sz, doff):
            acc_ref[pl.ds(row(dst_row), sz), :] = (
                acc_ref[pl.ds(row(dst_row), sz), :]
                + rcv_ref[pl.ds(doff, sz), :].astype(jnp.float32))

        layers = ((win0_ref, wout0_ref), (win1_ref, wout1_ref),
                  (win2_ref, wout2_ref))
        last = len(layers) - 1
        for l, (win_ref, wout_ref) in enumerate(layers):
            if l == 0:
                r1 = exchange(xf1_ref, my, AG_P_1[0](my), 0, H, ag1_s, ag1_r)
                r2 = exchange(xf2_ref, qmy, AG_P_2[0](my), 0, H, ag2_s, ag2_r)
                cast_weights(win_ref, wout_ref)
                r1.wait()
                r2.wait()
            r1 = exchange(xf1_ref, my, AG_P_1[1](my), 1, 2 * H, ag1_s, ag1_r)
            r2 = exchange(xf2_ref, qmy, AG_P_2[1](my), 1, 2 * H, ag2_s, ag2_r)
            c1 = row(lax.bitwise_and(my, 6) * H)
            c2 = row(lax.bitwise_and(qmy, 6) * H)
            compute_rows(xf1_ref, acc1_ref, c1, 2 * H)
            compute_rows(xf2_ref, acc2_ref, c2, 2 * H)
            r1.wait()
            r2.wait()
            r1 = exchange(xf1_ref, my, AG_P_1[2](my), 2, 4 * H, ag1_s, ag1_r)
            r2 = exchange(xf2_ref, qmy, AG_P_2[2](my), 2, 4 * H, ag2_s, ag2_r)
            compute_rows(xf1_ref, acc1_ref, lax.bitwise_xor(c1, 2 * H), 2 * H)
            compute_rows(xf2_ref, acc2_ref, lax.bitwise_xor(c2, 2 * H), 2 * H)
            r1.wait()
            r2.wait()

            oA1 = row(lax.bitwise_and(lax.bitwise_xor(my, 4), 6) * H)
            oA2 = row(lax.bitwise_and(lax.bitwise_xor(qmy, 4), 6) * H)
            compute_rows(xf1_ref, acc1_ref, oA1, 2 * H)
            compute_rows(xf2_ref, acc2_ref, oA2, 2 * H)
            d1a = send_block(oA1, 2 * H, RS_P_1[0](my), acc1_ref, snd1_ref,
                             rcv1_ref, rs1_s, rs1_r, 0, 0, 0)
            d2a = send_block(oA2, 2 * H, RS_P_2[0](my), acc2_ref, snd2_ref,
                             rcv2_ref, rs2_s, rs2_r, 0, 0, 0)
            compute_rows(xf1_ref, acc1_ref, lax.bitwise_xor(oA1, 2 * H), 2 * H)
            compute_rows(xf2_ref, acc2_ref, lax.bitwise_xor(oA2, 2 * H), 2 * H)
            d1b = send_block(lax.bitwise_xor(oA1, 2 * H), 2 * H,
                             RS_P_1[0](my), acc1_ref, snd1_ref, rcv1_ref,
                             rs1_s, rs1_r, 4, 2 * H, 2 * H)
            d2b = send_block(lax.bitwise_xor(oA2, 2 * H), 2 * H,
                             RS_P_2[0](my), acc2_ref, snd2_ref, rcv2_ref,
                             rs2_s, rs2_r, 4, 2 * H, 2 * H)
            if l < last:
                cast_weights(*layers[l + 1])
            d1a.wait()
            d2a.wait()
            add_in(acc1_ref, rcv1_ref, c1, 2 * H, 0)
            add_in(acc2_ref, rcv2_ref, c2, 2 * H, 0)
            d1b.wait()
            d2b.wait()
            add_in(acc1_ref, rcv1_ref, lax.bitwise_xor(c1, 2 * H), 2 * H,
                   2 * H)
            add_in(acc2_ref, rcv2_ref, lax.bitwise_xor(c2, 2 * H), 2 * H,
                   2 * H)

            s1 = lax.bitwise_xor(c1, 2 * H)
            s2 = lax.bitwise_xor(c2, 2 * H)
            d1 = send_block(s1, 2 * H, RS_P_1[1](my), acc1_ref, snd1_ref,
                            rcv1_ref, rs1_s, rs1_r, 1, 0, 4 * H)
            d2 = send_block(s2, 2 * H, RS_P_2[1](my), acc2_ref, snd2_ref,
                            rcv2_ref, rs2_s, rs2_r, 1, 0, 4 * H)
            d1.wait()
            d2.wait()
            add_in(acc1_ref, rcv1_ref, c1, 2 * H, 4 * H)
            add_in(acc2_ref, rcv2_ref, c2, 2 * H, 4 * H)

            if l < last:
                si = 2 + (l % 2)
                boff = (6 + 2 * (l % 2)) * H
                d1 = send_block(c1, 2 * H, RS_P_1[2](my), acc1_ref, snd1_ref,
                                rcv1_ref, rs1_s, rs1_r, si, 0, boff)
                d2 = send_block(c2, 2 * H, RS_P_2[2](my), acc2_ref, snd2_ref,
                                rcv2_ref, rs2_s, rs2_r, si, 0, boff)
                d1.wait()
                d2.wait()
                xf1_ref[pl.ds(c1, 2 * H), :] = (
                    acc1_ref[pl.ds(c1, 2 * H), :]
                    + rcv1_ref[pl.ds(boff, 2 * H), :].astype(jnp.float32)
                ).astype(jnp.bfloat16)
                xf2_ref[pl.ds(c2, 2 * H), :] = (
                    acc2_ref[pl.ds(c2, 2 * H), :]
                    + rcv2_ref[pl.ds(boff, 2 * H), :].astype(jnp.float32)
                ).astype(jnp.bfloat16)
            else:
                k1 = row(my * H)
                k2 = row(qmy * H)
                d1 = send_block(lax.bitwise_xor(k1, H), H, RS_P_1[2](my),
                                acc1_ref, snd1_ref, rcv1_ref, rs1_s, rs1_r,
                                2, 0, 6 * H)
                d2 = send_block(lax.bitwise_xor(k2, H), H, RS_P_2[2](my),
                                acc2_ref, snd2_ref, rcv2_ref, rs2_s, rs2_r,
                                2, 0, 6 * H)
                d1.wait()
                d2.wait()
                out_ref[0:H, :] = (
                    acc1_ref[pl.ds(k1, H), :]
                    + rcv1_ref[pl.ds(6 * H, H), :].astype(jnp.float32))
                out_ref[H:m_per, :] = (
                    acc2_ref[pl.ds(k2, H), :]
                    + rcv2_ref[pl.ds(6 * H, H), :].astype(jnp.float32))

    return pl.pallas_call(
        body,
        out_shape=jax.ShapeDtypeStruct((m_per, d), jnp.float32),
        in_specs=[pl.BlockSpec(memory_space=pltpu.VMEM)] * 7,
        out_specs=pl.BlockSpec(memory_space=pltpu.VMEM),
        scratch_shapes=[
            pltpu.VMEM((MH, d), jnp.bfloat16),
            pltpu.VMEM((MH, d), jnp.bfloat16),
            pltpu.VMEM((MH, d), jnp.float32),
            pltpu.VMEM((MH, d), jnp.float32),
            pltpu.VMEM((4 * H, d), jnp.bfloat16),
            pltpu.VMEM((4 * H, d), jnp.bfloat16),
            pltpu.VMEM((10 * H, d), jnp.bfloat16),
            pltpu.VMEM((10 * H, d), jnp.bfloat16),
            pltpu.VMEM((d, hid), jnp.bfloat16),
            pltpu.VMEM((hid, d), jnp.bfloat16),
            pltpu.SemaphoreType.DMA((3,)),
            pltpu.SemaphoreType.DMA((3,)),
            pltpu.SemaphoreType.DMA((3,)),
            pltpu.SemaphoreType.DMA((3,)),
            pltpu.SemaphoreType.DMA((5,)),
            pltpu.SemaphoreType.DMA((5,)),
            pltpu.SemaphoreType.DMA((5,)),
            pltpu.SemaphoreType.DMA((5,)),
        ],
        compiler_params=pltpu.CompilerParams(collective_id=0),
    )(x, Win0, Wout0, Win1, Wout1, Win2, Wout2)
